# baseline (speedup 1.0000x reference)
import functools

import jax
import jax.numpy as jnp
from jax import lax
from jax.experimental import pallas as pl
from jax.experimental.pallas import tpu as pltpu
from jax.experimental.pallas import tpu_sc as plsc

_TABLE_W = 128
_IDX_PAD = 24


def _mm(a, b):
    return lax.dot_general(a, b, (((a.ndim - 1,), (0,)), ((), ())),
                           preferred_element_type=jnp.float32)


def _mm_nt(a, b):
    return lax.dot_general(a, b, (((1,), (1,)), ((), ())),
                           preferred_element_type=jnp.float32)


def _make_stage1(N, L, k1):
    def body(featR, coordpR, coordtR, langR, lmaskR,
             fW1R, fb1R, fW2R, fb2R, lW1R, lb1R, lW2R, lb2R,
             idxR, tabR, fR):
        s = pl.program_id(0)

        t = jnp.maximum(_mm(featR[0], fW1R[...]) + fb1R[...], 0.0)
        f = _mm(t, fW2R[...]) + fb2R[...]

        t2 = jnp.maximum(_mm(langR[0], lW1R[...]) + lb1R[...], 0.0)
        lf = _mm(t2, lW2R[...]) + lb2R[...]

        sc = _mm_nt(f, lf)
        lane = lax.broadcasted_iota(jnp.int32, sc.shape, 1)
        sc = jnp.where(lane < L, sc, -jnp.inf)
        m = jnp.max(sc, axis=1, keepdims=True)
        e = jnp.exp(sc - m)
        p = e / jnp.sum(e, axis=1, keepdims=True)
        p = p * lmaskR[0]
        p = p / (jnp.sum(p, axis=1, keepdims=True) + 1e-7)
        ins = _mm(p, lf)

        fR[0] = f
        tabR[0, :, 0:16] = coordpR[0]
        tabR[0, :, 64:128] = f * ins

        cp = coordpR[0]
        ct = coordtR[0]
        d = None
        for c in range(3):
            diff = cp[:, c:c + 1] - ct[c:c + 1, :]
            d = diff * diff if d is None else d + diff * diff

        base = s * N
        big = jnp.float32(jnp.inf)

        ilane = lax.broadcasted_iota(jnp.int32, (N, 1), 0)
        idxR[0, :, 0:1] = ilane + base
        rl_i = lax.broadcasted_iota(jnp.int32, (N, N), 0)
        jl_i = lax.broadcasted_iota(jnp.int32, (N, N), 1)
        d = jnp.where(rl_i == jl_i, big, d)

        jlane = jl_i.astype(jnp.float32)
        nb = N // 128
        jl128 = lax.broadcasted_iota(jnp.int32, (N, 128), 1).astype(jnp.float32)
        for r in range(1, k1):
            m = d[:, 0:128]
            for b in range(1, nb):
                m = jnp.minimum(m, d[:, b * 128:(b + 1) * 128])
            mn = jnp.min(m, axis=1, keepdims=True)
            jc = jnp.full((N, 128), jnp.float32(N), jnp.float32)
            for b in range(nb):
                hit = jnp.where(d[:, b * 128:(b + 1) * 128] == mn,
                                jl128 + jnp.float32(b * 128), jnp.float32(N))
                jc = jnp.minimum(jc, hit)
            am = jnp.min(jc, axis=1, keepdims=True)
            idxR[0, :, r:r + 1] = am.astype(jnp.int32) + base
            d = jnp.where(jlane == am, big, d)

    return body


def _stage1_specs(S, N):
    bs = lambda shape: pl.BlockSpec(shape, lambda s: (s, 0, 0))
    w2 = lambda shape: pl.BlockSpec(shape, lambda s: (0, 0))
    return dict(
        grid=(S,),
        in_specs=[
            bs((1, N, 128)),
            bs((1, N, 16)),
            bs((1, 8, N)),
            bs((1, 64, 256)),
            bs((1, 1, 64)),
            w2((128, 64)), w2((1, 64)), w2((64, 64)), w2((1, 64)),
            w2((256, 64)), w2((1, 64)), w2((64, 64)), w2((1, 64)),
        ],
        out_specs=[
            bs((1, N, _IDX_PAD)),
            bs((1, N, _TABLE_W)),
            bs((1, N, 64)),
        ],
        out_shape=[
            jax.ShapeDtypeStruct((S, N, _IDX_PAD), jnp.int32),
            jax.ShapeDtypeStruct((S, N, _TABLE_W), jnp.float32),
            jax.ShapeDtypeStruct((S, N, 64), jnp.float32),
        ],
    )


def _make_gather(n_rows, width):
    info = plsc.get_sparse_core_info()
    nc, ns = info.num_cores, info.num_subcores
    nw = nc * ns
    bw = n_rows // nw
    ch = 128
    n_pairs = bw // (2 * ch)

    mesh = plsc.VectorSubcoreMesh(core_axis_name="c", subcore_axis_name="s")

    @functools.partial(
        pl.kernel, mesh=mesh,
        out_type=jax.ShapeDtypeStruct((n_rows, width), jnp.float32),
        scratch_types=[
            pltpu.VMEM((bw,), jnp.int32),
            pltpu.VMEM((ch, width), jnp.float32),
            pltpu.VMEM((ch, width), jnp.float32),
            pltpu.SemaphoreType.DMA,
            pltpu.SemaphoreType.DMA,
        ],
    )
    def gather_k(tab_hbm, idx_hbm, out_hbm, idx_v, buf0, buf1, sem0, sem1):
        wid = lax.axis_index("s") * nc + lax.axis_index("c")
        base = wid * bw
        pltpu.sync_copy(idx_hbm.at[pl.ds(base, bw)], idx_v)

        def step(p, carry):
            o0 = 2 * p * ch
            o1 = o0 + ch
            cp0 = pltpu.async_copy(
                tab_hbm.at[idx_v.at[pl.ds(o0, ch)]], buf0, sem0)
            cp1 = pltpu.async_copy(
                tab_hbm.at[idx_v.at[pl.ds(o1, ch)]], buf1, sem1)
            cp0.wait()
            pltpu.sync_copy(buf0, out_hbm.at[pl.ds(base + o0, ch)])
            cp1.wait()
            pltpu.sync_copy(buf1, out_hbm.at[pl.ds(base + o1, ch)])
            return carry

        lax.fori_loop(0, n_pairs, step, 0)

    return gather_k


def _make_stage2(N, k1):
    def body(hgR, coordpR, fR, rWacR, rWbcR, rwdrepR, rb1R, rW2R, rb2R,
             outR, scoreR):
        hg = hgR[0]
        cp = coordpR[0]
        m = k1 * N

        gc16 = hg[:, :, 0:16]
        rc16 = gc16 - cp[None]
        rc2 = (rc16 * rc16).reshape(m, 16)
        s3row = lax.broadcasted_iota(jnp.int32, (16, 16), 0)
        sum3 = jnp.where(s3row < 3, 1.0, 0.0)
        sq16 = _mm(rc2, sum3)
        dn16 = jnp.where(sq16 == 0.0, 0.0, jnp.sqrt(sq16))

        a_part = _mm(gc16.reshape(m, 16), rWacR[...])
        dnterm = _mm(dn16, rwdrepR[...])
        b_part = _mm(cp, rWbcR[...]) + rb1R[...]
        rel1 = (a_part + dnterm).reshape(k1, N, 64) + b_part[None]
        rel1 = jnp.maximum(rel1, 0.0)
        rel = (_mm(rel1.reshape(m, 64), rW2R[...]).reshape(k1, N, 64)
               + rb2R[...])

        acc = jnp.sum(hg[:, :, 64:128] * rel, axis=0) + fR[0]
        outR[0] = acc
        scoreR[0] = jnp.sum(acc, axis=1, keepdims=True)

    return body


def _stage2_specs(S, N, k1):
    bs3 = lambda shape: pl.BlockSpec(shape, lambda s: (s, 0, 0))
    w2 = lambda shape: pl.BlockSpec(shape, lambda s: (0, 0))
    return dict(
        grid=(S,),
        in_specs=[
            pl.BlockSpec((1, k1, N, _TABLE_W), lambda s: (s, 0, 0, 0)),
            bs3((1, N, 16)),
            bs3((1, N, 64)),
            w2((16, 64)), w2((16, 64)), w2((16, 64)), w2((1, 64)),
            w2((64, 64)), w2((1, 64)),
        ],
        out_specs=[
            bs3((1, N, 64)),
            bs3((1, N, 1)),
        ],
        out_shape=[
            jax.ShapeDtypeStruct((S, N, 64), jnp.float32),
            jax.ShapeDtypeStruct((S, N, 1), jnp.float32),
        ],
    )


def kernel(feat, coord, lang_feat, lang_mask,
           rW1, rb1, rW2, rb2, lW1, lb1, lW2, lb2, fW1, fb1, fW2, fb2):
    S, N, _ = feat.shape
    L = lang_feat.shape[1]
    k1 = min(16, N - 1) + 1

    coordp = jnp.pad(coord, ((0, 0), (0, 0), (0, 13)))
    coordt = jnp.pad(jnp.swapaxes(coord, 1, 2), ((0, 0), (0, 5), (0, 0)))
    langp = jnp.pad(lang_feat, ((0, 0), (0, 64 - L), (0, 0)))
    lmaskp = jnp.pad(lang_mask, ((0, 0), (0, 64 - L)))[:, None, :]
    r1 = lambda b: b.reshape(1, 64)

    zpad = jnp.zeros((13, 64), jnp.float32)
    rWac = jnp.concatenate([rW1[0:3] + rW1[6:9], zpad], axis=0)
    rWbc = jnp.concatenate([rW1[3:6] - rW1[6:9], zpad], axis=0)
    rwdrep = jnp.tile(rW1[9:10] * (1.0 / 16.0), (16, 1))

    ngrp = 4 if S % 4 == 0 else 1
    sg = S // ngrp
    stage1_call = pl.pallas_call(_make_stage1(N, L, k1), **_stage1_specs(sg, N))
    gather_call = _make_gather(sg * k1 * N, _TABLE_W)
    stage2_call = pl.pallas_call(_make_stage2(N, k1), **_stage2_specs(sg, N, k1))
    fos, scs = [], []
    for g in range(ngrp):
        sl = slice(g * sg, (g + 1) * sg)
        idx, table, f = stage1_call(
            feat[sl], coordp[sl], coordt[sl], langp[sl], lmaskp[sl],
            fW1, r1(fb1), fW2, r1(fb2), lW1, r1(lb1), lW2, r1(lb2))
        idx_flat = jnp.transpose(idx[:, :, :k1], (0, 2, 1)).reshape(sg * k1 * N)
        hg_flat = gather_call(table.reshape(sg * N, _TABLE_W), idx_flat)
        hg = hg_flat.reshape(sg, k1, N, _TABLE_W)
        fo, sc = stage2_call(hg, coordp[sl], f,
                             rWac, rWbc, rwdrep, r1(rb1), rW2, r1(rb2))
        fos.append(fo)
        scs.append(sc)
    feat_out = jnp.concatenate(fos, axis=0) if ngrp > 1 else fos[0]
    score = jnp.concatenate(scs, axis=0) if ngrp > 1 else scs[0]
    return feat_out, score.reshape(S, N)


# speedup vs baseline: 1.0382x; 1.0382x over previous
import functools

import jax
import jax.numpy as jnp
from jax import lax
from jax.experimental import pallas as pl
from jax.experimental.pallas import tpu as pltpu
from jax.experimental.pallas import tpu_sc as plsc

_TABLE_W = 128
_IDX_PAD = 24


def _mm(a, b):
    return lax.dot_general(a, b, (((a.ndim - 1,), (0,)), ((), ())),
                           preferred_element_type=jnp.float32)


def _mm_nt(a, b):
    return lax.dot_general(a, b, (((1,), (1,)), ((), ())),
                           preferred_element_type=jnp.float32)


def _make_stage1(N, L, k1):
    def body(featR, coordpR, coordtR, langR, lmaskR,
             fW1R, fb1R, fW2R, fb2R, lW1R, lb1R, lW2R, lb2R,
             idxR, tabR, fR):
        s = pl.program_id(0)

        t = jnp.maximum(_mm(featR[0], fW1R[...]) + fb1R[...], 0.0)
        f = _mm(t, fW2R[...]) + fb2R[...]

        t2 = jnp.maximum(_mm(langR[0], lW1R[...]) + lb1R[...], 0.0)
        lf = _mm(t2, lW2R[...]) + lb2R[...]

        sc = _mm_nt(f, lf)
        lane = lax.broadcasted_iota(jnp.int32, sc.shape, 1)
        sc = jnp.where(lane < L, sc, -jnp.inf)
        m = jnp.max(sc, axis=1, keepdims=True)
        e = jnp.exp(sc - m)
        p = e / jnp.sum(e, axis=1, keepdims=True)
        p = p * lmaskR[0]
        p = p / (jnp.sum(p, axis=1, keepdims=True) + 1e-7)
        ins = _mm(p, lf)

        fR[0] = f
        tabR[0, :, 0:16] = coordpR[0]
        tabR[0, :, 64:128] = f * ins

        cp = coordpR[0]
        ct = coordtR[0]
        d = None
        for c in range(3):
            diff = cp[:, c:c + 1] - ct[c:c + 1, :]
            d = diff * diff if d is None else d + diff * diff

        base = s * N
        big = jnp.float32(jnp.inf)

        ilane = lax.broadcasted_iota(jnp.int32, (N, 1), 0)
        idxR[0, :, 0:1] = ilane + base
        rl_i = lax.broadcasted_iota(jnp.int32, (N, N), 0)
        jl_i = lax.broadcasted_iota(jnp.int32, (N, N), 1)
        d = jnp.where(rl_i == jl_i, big, d)

        jlane = jl_i.astype(jnp.float32)
        nb = N // 128
        jl128 = lax.broadcasted_iota(jnp.int32, (N, 128), 1).astype(jnp.float32)
        for r in range(1, k1):
            m = d[:, 0:128]
            bb = jnp.zeros((N, 128), jnp.float32)
            for b in range(1, nb):
                cand = d[:, b * 128:(b + 1) * 128]
                lt = cand < m
                bb = jnp.where(lt, jnp.float32(b), bb)
                m = jnp.where(lt, cand, m)
            mn = jnp.min(m, axis=1, keepdims=True)
            jc = jnp.where(m == mn, bb * 128.0 + jl128, jnp.float32(N))
            am = jnp.min(jc, axis=1, keepdims=True)
            idxR[0, :, r:r + 1] = am.astype(jnp.int32) + base
            win = jc == am
            d = jnp.concatenate(
                [jnp.where(jnp.logical_and(win, bb == jnp.float32(b)), big,
                           d[:, b * 128:(b + 1) * 128])
                 for b in range(nb)], axis=1)

    return body


def _stage1_specs(S, N):
    bs = lambda shape: pl.BlockSpec(shape, lambda s: (s, 0, 0))
    w2 = lambda shape: pl.BlockSpec(shape, lambda s: (0, 0))
    return dict(
        grid=(S,),
        in_specs=[
            bs((1, N, 128)),
            bs((1, N, 16)),
            bs((1, 8, N)),
            bs((1, 64, 256)),
            bs((1, 1, 64)),
            w2((128, 64)), w2((1, 64)), w2((64, 64)), w2((1, 64)),
            w2((256, 64)), w2((1, 64)), w2((64, 64)), w2((1, 64)),
        ],
        out_specs=[
            bs((1, N, _IDX_PAD)),
            bs((1, N, _TABLE_W)),
            bs((1, N, 64)),
        ],
        out_shape=[
            jax.ShapeDtypeStruct((S, N, _IDX_PAD), jnp.int32),
            jax.ShapeDtypeStruct((S, N, _TABLE_W), jnp.float32),
            jax.ShapeDtypeStruct((S, N, 64), jnp.float32),
        ],
    )


def _make_gather(n_rows, width):
    info = plsc.get_sparse_core_info()
    nc, ns = info.num_cores, info.num_subcores
    nw = nc * ns
    bw = n_rows // nw
    ch = 128
    n_pairs = bw // (2 * ch)

    mesh = plsc.VectorSubcoreMesh(core_axis_name="c", subcore_axis_name="s")

    @functools.partial(
        pl.kernel, mesh=mesh,
        out_type=jax.ShapeDtypeStruct((n_rows, width), jnp.float32),
        scratch_types=[
            pltpu.VMEM((bw,), jnp.int32),
            pltpu.VMEM((ch, width), jnp.float32),
            pltpu.VMEM((ch, width), jnp.float32),
            pltpu.SemaphoreType.DMA,
            pltpu.SemaphoreType.DMA,
        ],
    )
    def gather_k(tab_hbm, idx_hbm, out_hbm, idx_v, buf0, buf1, sem0, sem1):
        wid = lax.axis_index("s") * nc + lax.axis_index("c")
        base = wid * bw
        pltpu.sync_copy(idx_hbm.at[pl.ds(base, bw)], idx_v)

        def step(p, carry):
            o0 = 2 * p * ch
            o1 = o0 + ch
            cp0 = pltpu.async_copy(
                tab_hbm.at[idx_v.at[pl.ds(o0, ch)]], buf0, sem0)
            cp1 = pltpu.async_copy(
                tab_hbm.at[idx_v.at[pl.ds(o1, ch)]], buf1, sem1)
            cp0.wait()
            pltpu.sync_copy(buf0, out_hbm.at[pl.ds(base + o0, ch)])
            cp1.wait()
            pltpu.sync_copy(buf1, out_hbm.at[pl.ds(base + o1, ch)])
            return carry

        lax.fori_loop(0, n_pairs, step, 0)

    return gather_k


def _make_stage2(N, k1):
    def body(hgR, coordpR, fR, rWacR, rWbcR, rwdrepR, rb1R, rW2R, rb2R,
             outR, scoreR):
        hg = hgR[0]
        cp = coordpR[0]
        m = k1 * N

        gc16 = hg[:, :, 0:16]
        rc16 = gc16 - cp[None]
        rc2 = (rc16 * rc16).reshape(m, 16)
        s3row = lax.broadcasted_iota(jnp.int32, (16, 16), 0)
        sum3 = jnp.where(s3row < 3, 1.0, 0.0)
        sq16 = _mm(rc2, sum3)
        dn16 = jnp.where(sq16 == 0.0, 0.0, jnp.sqrt(sq16))

        a_part = _mm(gc16.reshape(m, 16), rWacR[...])
        dnterm = _mm(dn16, rwdrepR[...])
        b_part = _mm(cp, rWbcR[...]) + rb1R[...]
        rel1 = (a_part + dnterm).reshape(k1, N, 64) + b_part[None]
        rel1 = jnp.maximum(rel1, 0.0)
        rel = (_mm(rel1.reshape(m, 64), rW2R[...]).reshape(k1, N, 64)
               + rb2R[...])

        acc = jnp.sum(hg[:, :, 64:128] * rel, axis=0) + fR[0]
        outR[0] = acc
        scoreR[0] = jnp.sum(acc, axis=1, keepdims=True)

    return body


def _stage2_specs(S, N, k1):
    bs3 = lambda shape: pl.BlockSpec(shape, lambda s: (s, 0, 0))
    w2 = lambda shape: pl.BlockSpec(shape, lambda s: (0, 0))
    return dict(
        grid=(S,),
        in_specs=[
            pl.BlockSpec((1, k1, N, _TABLE_W), lambda s: (s, 0, 0, 0)),
            bs3((1, N, 16)),
            bs3((1, N, 64)),
            w2((16, 64)), w2((16, 64)), w2((16, 64)), w2((1, 64)),
            w2((64, 64)), w2((1, 64)),
        ],
        out_specs=[
            bs3((1, N, 64)),
            bs3((1, N, 1)),
        ],
        out_shape=[
            jax.ShapeDtypeStruct((S, N, 64), jnp.float32),
            jax.ShapeDtypeStruct((S, N, 1), jnp.float32),
        ],
    )


def kernel(feat, coord, lang_feat, lang_mask,
           rW1, rb1, rW2, rb2, lW1, lb1, lW2, lb2, fW1, fb1, fW2, fb2):
    S, N, _ = feat.shape
    L = lang_feat.shape[1]
    k1 = min(16, N - 1) + 1

    coordp = jnp.pad(coord, ((0, 0), (0, 0), (0, 13)))
    coordt = jnp.pad(jnp.swapaxes(coord, 1, 2), ((0, 0), (0, 5), (0, 0)))
    langp = jnp.pad(lang_feat, ((0, 0), (0, 64 - L), (0, 0)))
    lmaskp = jnp.pad(lang_mask, ((0, 0), (0, 64 - L)))[:, None, :]
    r1 = lambda b: b.reshape(1, 64)

    zpad = jnp.zeros((13, 64), jnp.float32)
    rWac = jnp.concatenate([rW1[0:3] + rW1[6:9], zpad], axis=0)
    rWbc = jnp.concatenate([rW1[3:6] - rW1[6:9], zpad], axis=0)
    rwdrep = jnp.tile(rW1[9:10] * (1.0 / 16.0), (16, 1))

    ngrp = 4 if S % 4 == 0 else 1
    sg = S // ngrp
    stage1_call = pl.pallas_call(_make_stage1(N, L, k1), **_stage1_specs(sg, N))
    gather_call = _make_gather(sg * k1 * N, _TABLE_W)
    stage2_call = pl.pallas_call(_make_stage2(N, k1), **_stage2_specs(sg, N, k1))
    fos, scs = [], []
    for g in range(ngrp):
        sl = slice(g * sg, (g + 1) * sg)
        idx, table, f = stage1_call(
            feat[sl], coordp[sl], coordt[sl], langp[sl], lmaskp[sl],
            fW1, r1(fb1), fW2, r1(fb2), lW1, r1(lb1), lW2, r1(lb2))
        idx_flat = jnp.transpose(idx[:, :, :k1], (0, 2, 1)).reshape(sg * k1 * N)
        hg_flat = gather_call(table.reshape(sg * N, _TABLE_W), idx_flat)
        hg = hg_flat.reshape(sg, k1, N, _TABLE_W)
        fo, sc = stage2_call(hg, coordp[sl], f,
                             rWac, rWbc, rwdrep, r1(rb1), rW2, r1(rb2))
        fos.append(fo)
        scs.append(sc)
    feat_out = jnp.concatenate(fos, axis=0) if ngrp > 1 else fos[0]
    score = jnp.concatenate(scs, axis=0) if ngrp > 1 else scs[0]
    return feat_out, score.reshape(S, N)


# speedup vs baseline: 1.0617x; 1.0226x over previous
import functools

import jax
import jax.numpy as jnp
from jax import lax
from jax.experimental import pallas as pl
from jax.experimental.pallas import tpu as pltpu
from jax.experimental.pallas import tpu_sc as plsc

_TABLE_W = 128
_IDX_PAD = 24


def _mm(a, b):
    return lax.dot_general(a, b, (((a.ndim - 1,), (0,)), ((), ())),
                           preferred_element_type=jnp.float32)


def _mm_nt(a, b):
    return lax.dot_general(a, b, (((1,), (1,)), ((), ())),
                           preferred_element_type=jnp.float32)


def _make_stage1(N, L, k1):
    def body(featR, coordpR, coordtR, langR, lmaskR,
             fW1R, fb1R, fW2R, fb2R, lW1R, lb1R, lW2R, lb2R,
             idxR, tabR, fR):
        s = pl.program_id(0)

        t = jnp.maximum(_mm(featR[0], fW1R[...]) + fb1R[...], 0.0)
        f = _mm(t, fW2R[...]) + fb2R[...]

        t2 = jnp.maximum(_mm(langR[0], lW1R[...]) + lb1R[...], 0.0)
        lf = _mm(t2, lW2R[...]) + lb2R[...]

        sc = _mm_nt(f, lf)
        lane = lax.broadcasted_iota(jnp.int32, sc.shape, 1)
        sc = jnp.where(lane < L, sc, -jnp.inf)
        m = jnp.max(sc, axis=1, keepdims=True)
        e = jnp.exp(sc - m)
        p = e / jnp.sum(e, axis=1, keepdims=True)
        p = p * lmaskR[0]
        p = p / (jnp.sum(p, axis=1, keepdims=True) + 1e-7)
        ins = _mm(p, lf)

        fR[0] = f
        tabR[0, :, 0:16] = coordpR[0]
        tabR[0, :, 64:128] = f * ins

        cp = coordpR[0]
        ct = coordtR[0]
        d = None
        for c in range(3):
            diff = cp[:, c:c + 1] - ct[c:c + 1, :]
            d = diff * diff if d is None else d + diff * diff

        base = s * N
        big = jnp.float32(jnp.inf)

        ilane = lax.broadcasted_iota(jnp.int32, (N, 1), 0)
        idxR[0, :, 0:1] = ilane + base
        rl_i = lax.broadcasted_iota(jnp.int32, (N, N), 0)
        jl_i = lax.broadcasted_iota(jnp.int32, (N, N), 1)
        d = jnp.where(rl_i == jl_i, big, d)

        jlane = jl_i.astype(jnp.float32)
        nb = N // 128
        jl128 = lax.broadcasted_iota(jnp.int32, (N, 128), 1).astype(jnp.float32)
        for r in range(1, k1):
            m = d[:, 0:128]
            bb = jnp.zeros((N, 128), jnp.float32)
            for b in range(1, nb):
                cand = d[:, b * 128:(b + 1) * 128]
                lt = cand < m
                bb = jnp.where(lt, jnp.float32(b), bb)
                m = jnp.where(lt, cand, m)
            mn = jnp.min(m, axis=1, keepdims=True)
            jc = jnp.where(m == mn, bb * 128.0 + jl128, jnp.float32(N))
            am = jnp.min(jc, axis=1, keepdims=True)
            idxR[0, :, r:r + 1] = am.astype(jnp.int32) + base
            d = jnp.where(jlane == am, big, d)

    return body


def _stage1_specs(S, N):
    bs = lambda shape: pl.BlockSpec(shape, lambda s: (s, 0, 0))
    w2 = lambda shape: pl.BlockSpec(shape, lambda s: (0, 0))
    return dict(
        grid=(S,),
        in_specs=[
            bs((1, N, 128)),
            bs((1, N, 16)),
            bs((1, 8, N)),
            bs((1, 64, 256)),
            bs((1, 1, 64)),
            w2((128, 64)), w2((1, 64)), w2((64, 64)), w2((1, 64)),
            w2((256, 64)), w2((1, 64)), w2((64, 64)), w2((1, 64)),
        ],
        out_specs=[
            bs((1, N, _IDX_PAD)),
            bs((1, N, _TABLE_W)),
            bs((1, N, 64)),
        ],
        out_shape=[
            jax.ShapeDtypeStruct((S, N, _IDX_PAD), jnp.int32),
            jax.ShapeDtypeStruct((S, N, _TABLE_W), jnp.float32),
            jax.ShapeDtypeStruct((S, N, 64), jnp.float32),
        ],
    )


def _make_gather(n_rows, width):
    info = plsc.get_sparse_core_info()
    nc, ns = info.num_cores, info.num_subcores
    nw = nc * ns
    bw = n_rows // nw
    ch = 128
    n_pairs = bw // (2 * ch)

    mesh = plsc.VectorSubcoreMesh(core_axis_name="c", subcore_axis_name="s")

    @functools.partial(
        pl.kernel, mesh=mesh,
        out_type=jax.ShapeDtypeStruct((n_rows, width), jnp.float32),
        scratch_types=[
            pltpu.VMEM((bw,), jnp.int32),
            pltpu.VMEM((ch, width), jnp.float32),
            pltpu.VMEM((ch, width), jnp.float32),
            pltpu.SemaphoreType.DMA,
            pltpu.SemaphoreType.DMA,
        ],
    )
    def gather_k(tab_hbm, idx_hbm, out_hbm, idx_v, buf0, buf1, sem0, sem1):
        wid = lax.axis_index("s") * nc + lax.axis_index("c")
        base = wid * bw
        pltpu.sync_copy(idx_hbm.at[pl.ds(base, bw)], idx_v)

        def step(p, carry):
            o0 = 2 * p * ch
            o1 = o0 + ch
            cp0 = pltpu.async_copy(
                tab_hbm.at[idx_v.at[pl.ds(o0, ch)]], buf0, sem0)
            cp1 = pltpu.async_copy(
                tab_hbm.at[idx_v.at[pl.ds(o1, ch)]], buf1, sem1)
            cp0.wait()
            pltpu.sync_copy(buf0, out_hbm.at[pl.ds(base + o0, ch)])
            cp1.wait()
            pltpu.sync_copy(buf1, out_hbm.at[pl.ds(base + o1, ch)])
            return carry

        lax.fori_loop(0, n_pairs, step, 0)

    return gather_k


def _make_stage2(N, k1):
    def body(hgR, coordpR, fR, rWacR, rWbcR, rwdrepR, rb1R, rW2R, rb2R,
             outR, scoreR):
        hg = hgR[0]
        cp = coordpR[0]
        m = k1 * N

        gc16 = hg[:, :, 0:16]
        rc16 = gc16 - cp[None]
        rc2 = (rc16 * rc16).reshape(m, 16)
        s3row = lax.broadcasted_iota(jnp.int32, (16, 16), 0)
        sum3 = jnp.where(s3row < 3, 1.0, 0.0)
        sq16 = _mm(rc2, sum3)
        dn16 = jnp.where(sq16 == 0.0, 0.0, jnp.sqrt(sq16))

        a_part = _mm(gc16.reshape(m, 16), rWacR[...])
        dnterm = _mm(dn16, rwdrepR[...])
        b_part = _mm(cp, rWbcR[...]) + rb1R[...]
        rel1 = (a_part + dnterm).reshape(k1, N, 64) + b_part[None]
        rel1 = jnp.maximum(rel1, 0.0)
        rel = (_mm(rel1.reshape(m, 64), rW2R[...]).reshape(k1, N, 64)
               + rb2R[...])

        acc = jnp.sum(hg[:, :, 64:128] * rel, axis=0) + fR[0]
        outR[0] = acc
        scoreR[0] = jnp.sum(acc, axis=1, keepdims=True)

    return body


def _stage2_specs(S, N, k1):
    bs3 = lambda shape: pl.BlockSpec(shape, lambda s: (s, 0, 0))
    w2 = lambda shape: pl.BlockSpec(shape, lambda s: (0, 0))
    return dict(
        grid=(S,),
        in_specs=[
            pl.BlockSpec((1, k1, N, _TABLE_W), lambda s: (s, 0, 0, 0)),
            bs3((1, N, 16)),
            bs3((1, N, 64)),
            w2((16, 64)), w2((16, 64)), w2((16, 64)), w2((1, 64)),
            w2((64, 64)), w2((1, 64)),
        ],
        out_specs=[
            bs3((1, N, 64)),
            bs3((1, N, 1)),
        ],
        out_shape=[
            jax.ShapeDtypeStruct((S, N, 64), jnp.float32),
            jax.ShapeDtypeStruct((S, N, 1), jnp.float32),
        ],
    )


def kernel(feat, coord, lang_feat, lang_mask,
           rW1, rb1, rW2, rb2, lW1, lb1, lW2, lb2, fW1, fb1, fW2, fb2):
    S, N, _ = feat.shape
    L = lang_feat.shape[1]
    k1 = min(16, N - 1) + 1

    coordp = jnp.pad(coord, ((0, 0), (0, 0), (0, 13)))
    coordt = jnp.pad(jnp.swapaxes(coord, 1, 2), ((0, 0), (0, 5), (0, 0)))
    langp = jnp.pad(lang_feat, ((0, 0), (0, 64 - L), (0, 0)))
    lmaskp = jnp.pad(lang_mask, ((0, 0), (0, 64 - L)))[:, None, :]
    r1 = lambda b: b.reshape(1, 64)

    zpad = jnp.zeros((13, 64), jnp.float32)
    rWac = jnp.concatenate([rW1[0:3] + rW1[6:9], zpad], axis=0)
    rWbc = jnp.concatenate([rW1[3:6] - rW1[6:9], zpad], axis=0)
    rwdrep = jnp.tile(rW1[9:10] * (1.0 / 16.0), (16, 1))

    ngrp = 4 if S % 4 == 0 else 1
    sg = S // ngrp
    stage1_call = pl.pallas_call(_make_stage1(N, L, k1), **_stage1_specs(sg, N))
    gather_call = _make_gather(sg * k1 * N, _TABLE_W)
    stage2_call = pl.pallas_call(_make_stage2(N, k1), **_stage2_specs(sg, N, k1))
    fos, scs = [], []
    for g in range(ngrp):
        sl = slice(g * sg, (g + 1) * sg)
        idx, table, f = stage1_call(
            feat[sl], coordp[sl], coordt[sl], langp[sl], lmaskp[sl],
            fW1, r1(fb1), fW2, r1(fb2), lW1, r1(lb1), lW2, r1(lb2))
        idx_flat = jnp.transpose(idx[:, :, :k1], (0, 2, 1)).reshape(sg * k1 * N)
        hg_flat = gather_call(table.reshape(sg * N, _TABLE_W), idx_flat)
        hg = hg_flat.reshape(sg, k1, N, _TABLE_W)
        fo, sc = stage2_call(hg, coordp[sl], f,
                             rWac, rWbc, rwdrep, r1(rb1), rW2, r1(rb2))
        fos.append(fo)
        scs.append(sc)
    feat_out = jnp.concatenate(fos, axis=0) if ngrp > 1 else fos[0]
    score = jnp.concatenate(scs, axis=0) if ngrp > 1 else scs[0]
    return feat_out, score.reshape(S, N)
